# baseline (speedup 1.0000x reference)
import functools

import jax
import jax.numpy as jnp
from jax import lax
from jax.experimental import pallas as pl
from jax.experimental.pallas import tpu as pltpu
from jax.experimental.pallas import tpu_sc as plsc

B = 16384
D = 256
LANES = 16
NC = 2
NS = 16
NW = NC * NS
BPW = B // NW
C = 64
NCH = BPW // C
KCH = D // LANES


def _dot_chunk(ubuf, ibuf, xui_v, slot, c):
    lane = lax.iota(jnp.int32, LANES)
    slotv = jnp.full((LANES,), slot, jnp.int32)
    zero = jnp.zeros((LANES,), jnp.float32)

    def group_body(g, _):
        rows = g * LANES + lane

        def col_body(k, accs):
            out = []
            for t, a in enumerate(accs):
                kv = jnp.full((LANES,), k * 4 + t, jnp.int32)
                u = plsc.load_gather(ubuf, [slotv, rows, kv])
                v = plsc.load_gather(ibuf, [slotv, rows, kv])
                out.append(a + u * v)
            return tuple(out)

        a = lax.fori_loop(0, D // 4, col_body, (zero, zero, zero, zero),
                          unroll=2)
        xui_v[pl.ds(c * C + g * LANES, LANES)] = (a[0] + a[1]) + (a[2] + a[3])
        return 0

    lax.fori_loop(0, C // LANES, group_body, 0)


@functools.partial(
    pl.kernel,
    out_type=(
        jax.ShapeDtypeStruct((B,), jnp.float32),
        jax.ShapeDtypeStruct((B, D), jnp.float32),
        jax.ShapeDtypeStruct((B, D), jnp.float32),
    ),
    mesh=plsc.VectorSubcoreMesh(core_axis_name="c", subcore_axis_name="s"),
    compiler_params=pltpu.CompilerParams(use_tc_tiling_on_sc=False,
                                         needs_layout_passes=False),
    scratch_types=[
        pltpu.VMEM((BPW,), jnp.int32),
        pltpu.VMEM((BPW,), jnp.int32),
        pltpu.VMEM((2, C, D), jnp.float32),
        pltpu.VMEM((2, C, D), jnp.float32),
        pltpu.VMEM((BPW,), jnp.float32),
        pltpu.SemaphoreType.DMA,
        pltpu.SemaphoreType.DMA,
        pltpu.SemaphoreType.DMA,
        pltpu.SemaphoreType.DMA,
    ],
)
def _ngcf_sc(user_h, item_h, Gu_h, Gi_h, xui_o, gu_o, gi_o,
             uidx, iidx, ubuf, ibuf, xui_v, gsem0, gsem1, wsem0, wsem1):
    wid = lax.axis_index("s") * NC + lax.axis_index("c")
    base = wid * BPW

    pltpu.sync_copy(user_h.at[pl.ds(base, BPW)], uidx)
    pltpu.sync_copy(item_h.at[pl.ds(base, BPW)], iidx)

    gsems = (gsem0, gsem1)
    wsems = (wsem0, wsem1)

    def start_gather(c):
        slot = c % 2
        cu = pltpu.async_copy(Gu_h.at[uidx.at[pl.ds(c * C, C)]],
                              ubuf.at[slot], gsems[slot])
        ci = pltpu.async_copy(Gi_h.at[iidx.at[pl.ds(c * C, C)]],
                              ibuf.at[slot], gsems[slot])
        return cu, ci

    pend = {0: start_gather(0), 1: start_gather(1)}
    tail = {}
    for c in range(NCH):
        slot = c % 2
        cu, ci = pend.pop(c)
        cu.wait()
        ci.wait()
        wu = pltpu.async_copy(ubuf.at[slot], gu_o.at[pl.ds(base + c * C, C)],
                              wsems[slot])
        wi = pltpu.async_copy(ibuf.at[slot], gi_o.at[pl.ds(base + c * C, C)],
                              wsems[slot])
        _dot_chunk(ubuf, ibuf, xui_v, slot, c)
        if c + 2 < NCH:
            wu.wait()
            wi.wait()
            pend[c + 2] = start_gather(c + 2)
        else:
            tail[c] = (wu, wi)

    for wu, wi in tail.values():
        wu.wait()
        wi.wait()
    pltpu.sync_copy(xui_v, xui_o.at[pl.ds(base, BPW)])


def kernel(user, item, Gu, Gi):
    xui, gamma_u, gamma_i = _ngcf_sc(user, item, Gu, Gi)
    return (xui, gamma_u, gamma_i)


# speedup vs baseline: 2.0568x; 2.0568x over previous
import functools

import jax
import jax.numpy as jnp
from jax import lax
from jax.experimental import pallas as pl
from jax.experimental.pallas import tpu as pltpu
from jax.experimental.pallas import tpu_sc as plsc

B = 16384
D = 256
LANES = 16
NC = 2
NS = 16
NW = NC * NS
BPW = B // NW
C = 64
NCH = BPW // C
KCH = D // LANES


def _dot_chunk(ubuf, ibuf, xui_v, slot, c):
    lane = lax.iota(jnp.int32, LANES)
    slotv = jnp.full((LANES,), slot, jnp.int32)
    zero = jnp.zeros((LANES,), jnp.float32)

    def group_body(g, _):
        rows = g * LANES + lane

        def col_body(k, accs):
            out = []
            for t, a in enumerate(accs):
                kv = jnp.full((LANES,), k * 4 + t, jnp.int32)
                u = plsc.load_gather(ubuf, [slotv, rows, kv])
                v = plsc.load_gather(ibuf, [slotv, rows, kv])
                out.append(a + u * v)
            return tuple(out)

        a = lax.fori_loop(0, D // 4, col_body, (zero, zero, zero, zero),
                          unroll=2)
        xui_v[pl.ds(c * C + g * LANES, LANES)] = (a[0] + a[1]) + (a[2] + a[3])
        return 0

    lax.fori_loop(0, C // LANES, group_body, 0)


@functools.partial(
    pl.kernel,
    out_type=(
        jax.ShapeDtypeStruct((B,), jnp.float32),
        jax.ShapeDtypeStruct((B, D), jnp.float32),
        jax.ShapeDtypeStruct((B, D), jnp.float32),
    ),
    mesh=plsc.VectorSubcoreMesh(core_axis_name="c", subcore_axis_name="s"),
    compiler_params=pltpu.CompilerParams(needs_layout_passes=False),
    scratch_types=[
        pltpu.VMEM((BPW,), jnp.int32),
        pltpu.VMEM((BPW,), jnp.int32),
        pltpu.VMEM((2, C, D), jnp.float32),
        pltpu.VMEM((2, C, D), jnp.float32),
        pltpu.VMEM((BPW,), jnp.float32),
        pltpu.SemaphoreType.DMA,
        pltpu.SemaphoreType.DMA,
        pltpu.SemaphoreType.DMA,
        pltpu.SemaphoreType.DMA,
    ],
)
def _ngcf_sc(user_h, item_h, Gu_h, Gi_h, xui_o, gu_o, gi_o,
             uidx, iidx, ubuf, ibuf, xui_v, gsem0, gsem1, wsem0, wsem1):
    wid = lax.axis_index("s") * NC + lax.axis_index("c")
    base = wid * BPW

    pltpu.sync_copy(user_h.at[pl.ds(base, BPW)], uidx)
    pltpu.sync_copy(item_h.at[pl.ds(base, BPW)], iidx)

    gsems = (gsem0, gsem1)
    wsems = (wsem0, wsem1)

    def start_gather(c):
        slot = c % 2
        cu = pltpu.async_copy(Gu_h.at[uidx.at[pl.ds(c * C, C)]],
                              ubuf.at[slot], gsems[slot])
        ci = pltpu.async_copy(Gi_h.at[iidx.at[pl.ds(c * C, C)]],
                              ibuf.at[slot], gsems[slot])
        return cu, ci

    pend = {0: start_gather(0), 1: start_gather(1)}
    tail = {}
    for c in range(NCH):
        slot = c % 2
        cu, ci = pend.pop(c)
        cu.wait()
        ci.wait()
        wu = pltpu.async_copy(ubuf.at[slot], gu_o.at[pl.ds(base + c * C, C)],
                              wsems[slot])
        wi = pltpu.async_copy(ibuf.at[slot], gi_o.at[pl.ds(base + c * C, C)],
                              wsems[slot])
        _dot_chunk(ubuf, ibuf, xui_v, slot, c)
        if c + 2 < NCH:
            wu.wait()
            wi.wait()
            pend[c + 2] = start_gather(c + 2)
        else:
            tail[c] = (wu, wi)

    for wu, wi in tail.values():
        wu.wait()
        wi.wait()
    pltpu.sync_copy(xui_v, xui_o.at[pl.ds(base, BPW)])


def kernel(user, item, Gu, Gi):
    xui, gamma_u, gamma_i = _ngcf_sc(user, item, Gu, Gi)
    return (xui, gamma_u, gamma_i)


# speedup vs baseline: 4.9838x; 2.4231x over previous
import functools

import jax
import jax.numpy as jnp
from jax import lax
from jax.experimental import pallas as pl
from jax.experimental.pallas import tpu as pltpu
from jax.experimental.pallas import tpu_sc as plsc

B = 16384
D = 256
LANES = 16
NC = 2
NS = 16
NW = NC * NS
BPW = B // NW
C = 64
NCH = BPW // C
KCH = D // LANES


def _dot_chunk(ubuf, ibuf, xui_v, slot, c):
    lane = lax.iota(jnp.int32, LANES)

    def group_body(g, _):
        vals = jnp.zeros((LANES,), jnp.float32)
        for j in range(LANES):
            r = g * LANES + j
            acc = ubuf[slot, r, pl.ds(0, LANES)] * ibuf[slot, r, pl.ds(0, LANES)]
            for k in range(1, KCH):
                acc = acc + (ubuf[slot, r, pl.ds(k * LANES, LANES)]
                             * ibuf[slot, r, pl.ds(k * LANES, LANES)])
            vals = jnp.where(lane == j, jnp.sum(acc), vals)
        xui_v[pl.ds(c * C + g * LANES, LANES)] = vals
        return 0

    lax.fori_loop(0, C // LANES, group_body, 0)


@functools.partial(
    pl.kernel,
    out_type=(
        jax.ShapeDtypeStruct((B,), jnp.float32),
        jax.ShapeDtypeStruct((B, D), jnp.float32),
        jax.ShapeDtypeStruct((B, D), jnp.float32),
    ),
    mesh=plsc.VectorSubcoreMesh(core_axis_name="c", subcore_axis_name="s"),
    compiler_params=pltpu.CompilerParams(needs_layout_passes=False),
    scratch_types=[
        pltpu.VMEM((BPW,), jnp.int32),
        pltpu.VMEM((BPW,), jnp.int32),
        pltpu.VMEM((2, C, D), jnp.float32),
        pltpu.VMEM((2, C, D), jnp.float32),
        pltpu.VMEM((BPW,), jnp.float32),
        pltpu.SemaphoreType.DMA,
        pltpu.SemaphoreType.DMA,
        pltpu.SemaphoreType.DMA,
        pltpu.SemaphoreType.DMA,
    ],
)
def _ngcf_sc(user_h, item_h, Gu_h, Gi_h, xui_o, gu_o, gi_o,
             uidx, iidx, ubuf, ibuf, xui_v, gsem0, gsem1, wsem0, wsem1):
    wid = lax.axis_index("s") * NC + lax.axis_index("c")
    base = wid * BPW

    pltpu.sync_copy(user_h.at[pl.ds(base, BPW)], uidx)
    pltpu.sync_copy(item_h.at[pl.ds(base, BPW)], iidx)

    gsems = (gsem0, gsem1)
    wsems = (wsem0, wsem1)

    def start_gather(c):
        slot = c % 2
        cu = pltpu.async_copy(Gu_h.at[uidx.at[pl.ds(c * C, C)]],
                              ubuf.at[slot], gsems[slot])
        ci = pltpu.async_copy(Gi_h.at[iidx.at[pl.ds(c * C, C)]],
                              ibuf.at[slot], gsems[slot])
        return cu, ci

    pend = {0: start_gather(0), 1: start_gather(1)}
    tail = {}
    for c in range(NCH):
        slot = c % 2
        cu, ci = pend.pop(c)
        cu.wait()
        ci.wait()
        wu = pltpu.async_copy(ubuf.at[slot], gu_o.at[pl.ds(base + c * C, C)],
                              wsems[slot])
        wi = pltpu.async_copy(ibuf.at[slot], gi_o.at[pl.ds(base + c * C, C)],
                              wsems[slot])
        _dot_chunk(ubuf, ibuf, xui_v, slot, c)
        if c + 2 < NCH:
            wu.wait()
            wi.wait()
            pend[c + 2] = start_gather(c + 2)
        else:
            tail[c] = (wu, wi)

    for wu, wi in tail.values():
        wu.wait()
        wi.wait()
    pltpu.sync_copy(xui_v, xui_o.at[pl.ds(base, BPW)])


def kernel(user, item, Gu, Gi):
    xui, gamma_u, gamma_i = _ngcf_sc(user, item, Gu, Gi)
    return (xui, gamma_u, gamma_i)


# speedup vs baseline: 5.9311x; 1.1901x over previous
import functools

import jax
import jax.numpy as jnp
from jax import lax
from jax.experimental import pallas as pl
from jax.experimental.pallas import tpu as pltpu
from jax.experimental.pallas import tpu_sc as plsc

B = 16384
D = 256
LANES = 16
NC = 2
NS = 16
NW = NC * NS
BPW = B // NW
C = 64
NCH = BPW // C
NSLOT = 3
KCH = D // LANES


def _dot_chunk(ubuf, ibuf, xui_v, slot, c):
    lane = lax.iota(jnp.int32, LANES)

    def group_body(g, _):
        def row_body(j, vals):
            r = g * LANES + j
            acc = ubuf[slot, r, pl.ds(0, LANES)] * ibuf[slot, r, pl.ds(0, LANES)]
            for k in range(1, KCH):
                acc = acc + (ubuf[slot, r, pl.ds(k * LANES, LANES)]
                             * ibuf[slot, r, pl.ds(k * LANES, LANES)])
            return jnp.where(lane == j, jnp.sum(acc), vals)

        vals = lax.fori_loop(0, LANES, row_body,
                             jnp.zeros((LANES,), jnp.float32), unroll=4)
        xui_v[pl.ds(c * C + g * LANES, LANES)] = vals
        return 0

    lax.fori_loop(0, C // LANES, group_body, 0)


@functools.partial(
    pl.kernel,
    out_type=(
        jax.ShapeDtypeStruct((B,), jnp.float32),
        jax.ShapeDtypeStruct((B, D), jnp.float32),
        jax.ShapeDtypeStruct((B, D), jnp.float32),
    ),
    mesh=plsc.VectorSubcoreMesh(core_axis_name="c", subcore_axis_name="s"),
    compiler_params=pltpu.CompilerParams(needs_layout_passes=False),
    scratch_types=[
        pltpu.VMEM((BPW,), jnp.int32),
        pltpu.VMEM((BPW,), jnp.int32),
        pltpu.VMEM((NSLOT, C, D), jnp.float32),
        pltpu.VMEM((NSLOT, C, D), jnp.float32),
        pltpu.VMEM((BPW,), jnp.float32),
        pltpu.SemaphoreType.DMA,
        pltpu.SemaphoreType.DMA,
        pltpu.SemaphoreType.DMA,
        pltpu.SemaphoreType.DMA,
        pltpu.SemaphoreType.DMA,
        pltpu.SemaphoreType.DMA,
    ],
)
def _ngcf_sc(user_h, item_h, Gu_h, Gi_h, xui_o, gu_o, gi_o,
             uidx, iidx, ubuf, ibuf, xui_v,
             gsem0, gsem1, gsem2, wsem0, wsem1, wsem2):
    wid = lax.axis_index("s") * NC + lax.axis_index("c")
    base = wid * BPW

    pltpu.sync_copy(user_h.at[pl.ds(base, BPW)], uidx)
    pltpu.sync_copy(item_h.at[pl.ds(base, BPW)], iidx)

    gsems = (gsem0, gsem1, gsem2)
    wsems = (wsem0, wsem1, wsem2)

    def start_gather(c):
        slot = c % NSLOT
        cu = pltpu.async_copy(Gu_h.at[uidx.at[pl.ds(c * C, C)]],
                              ubuf.at[slot], gsems[slot])
        ci = pltpu.async_copy(Gi_h.at[iidx.at[pl.ds(c * C, C)]],
                              ibuf.at[slot], gsems[slot])
        return cu, ci

    def start_write(c):
        slot = c % NSLOT
        wu = pltpu.async_copy(ubuf.at[slot], gu_o.at[pl.ds(base + c * C, C)],
                              wsems[slot])
        wi = pltpu.async_copy(ibuf.at[slot], gi_o.at[pl.ds(base + c * C, C)],
                              wsems[slot])
        return wu, wi

    gpend = {0: start_gather(0), 1: start_gather(1)}
    wpend = {}
    for c in range(NCH):
        cu, ci = gpend.pop(c)
        cu.wait()
        ci.wait()
        wpend[c] = start_write(c)
        if c + 2 < NCH:
            prev = c - 1
            if prev in wpend:
                wu, wi = wpend.pop(prev)
                wu.wait()
                wi.wait()
            gpend[c + 2] = start_gather(c + 2)
        _dot_chunk(ubuf, ibuf, xui_v, c % NSLOT, c)

    for wu, wi in wpend.values():
        wu.wait()
        wi.wait()
    pltpu.sync_copy(xui_v, xui_o.at[pl.ds(base, BPW)])


def kernel(user, item, Gu, Gi):
    xui, gamma_u, gamma_i = _ngcf_sc(user, item, Gu, Gi)
    return (xui, gamma_u, gamma_i)
